# Initial kernel scaffold; baseline (speedup 1.0000x reference)
import jax
import jax.numpy as jnp
from jax.experimental import pallas as pl


def kernel(event, table):
    raise NotImplementedError("write your pallas kernel here")


# speedup vs baseline: 4.7997x; 4.7997x over previous
import functools

import jax
import jax.numpy as jnp
from jax import lax
from jax.experimental import pallas as pl
from jax.experimental.pallas import tpu as pltpu
from jax.experimental.pallas import tpu_sc as plsc

D_MODEL = 32
NC = 2
NS = 16
NW = NC * NS
CHUNK = 1024
KSTREAMS = CHUNK // 128


@functools.lru_cache(maxsize=None)
def _make_gather(B: int):
    b_per_w = B // NW
    n_chunks = b_per_w // CHUNK
    rows_per_w = b_per_w // 128
    mesh = plsc.VectorSubcoreMesh(core_axis_name="c", subcore_axis_name="s")

    @functools.partial(
        pl.kernel,
        mesh=mesh,
        compiler_params=pltpu.CompilerParams(use_tc_tiling_on_sc=False),
        out_type=jax.ShapeDtypeStruct((B, D_MODEL), jnp.float32),
        scratch_types=[
            pltpu.VMEM((KSTREAMS, 128), jnp.int32),
            pltpu.VMEM((CHUNK, D_MODEL), jnp.float32),
            pltpu.SemaphoreType.DMA,
        ],
    )
    def gather_kernel(idx_hbm, table_hbm, out_hbm, idx_v, rows_v, sem):
        wid = lax.axis_index("s") * NC + lax.axis_index("c")
        row0 = wid * rows_per_w
        out0 = wid * b_per_w

        def body(c, carry):
            pltpu.sync_copy(idx_hbm.at[pl.ds(row0 + c * KSTREAMS, KSTREAMS)],
                            idx_v)
            copies = []
            for j in range(KSTREAMS):
                copies.append(pltpu.async_copy(
                    table_hbm.at[idx_v.at[j]],
                    rows_v.at[pl.ds(j * 128, 128)],
                    sem))
            for cp in copies:
                cp.wait()
            pltpu.sync_copy(rows_v, out_hbm.at[pl.ds(out0 + c * CHUNK, CHUNK)])
            return carry

        lax.fori_loop(0, n_chunks, body, 0)

    return gather_kernel


def kernel(event, table):
    n, s = event.shape
    B = n * s
    idx2d = event.reshape(B // 128, 128)
    out = _make_gather(B)(idx2d, table)
    return out.reshape(n, s, D_MODEL)


# speedup vs baseline: 4.8952x; 1.0199x over previous
import functools

import jax
import jax.numpy as jnp
from jax import lax
from jax.experimental import pallas as pl
from jax.experimental.pallas import tpu as pltpu
from jax.experimental.pallas import tpu_sc as plsc

D_MODEL = 32
NC = 2
NS = 16
NW = NC * NS
RCHUNK = 8


@functools.lru_cache(maxsize=None)
def _make_gather(N: int, S: int):
    rows_per_w = N // NW
    n_chunks = rows_per_w // RCHUNK
    splits = []
    off = 0
    while off < S:
        n = min(128, S - off)
        splits.append((off, n))
        off += n
    mesh = plsc.VectorSubcoreMesh(core_axis_name="c", subcore_axis_name="s")

    @functools.partial(
        pl.kernel,
        mesh=mesh,
        compiler_params=pltpu.CompilerParams(use_tc_tiling_on_sc=False),
        out_type=jax.ShapeDtypeStruct((N, S, D_MODEL), jnp.float32),
        scratch_types=[
            pltpu.VMEM((RCHUNK, S), jnp.int32),
            pltpu.VMEM((RCHUNK, S, D_MODEL), jnp.float32),
            pltpu.SemaphoreType.DMA,
        ],
    )
    def gather_kernel(event_hbm, table_hbm, out_hbm, idx_v, rows_v, sem):
        wid = lax.axis_index("s") * NC + lax.axis_index("c")
        row0 = wid * rows_per_w

        def body(c, carry):
            r = row0 + c * RCHUNK
            pltpu.sync_copy(event_hbm.at[pl.ds(r, RCHUNK)], idx_v)
            copies = []
            for j in range(RCHUNK):
                for off, n in splits:
                    copies.append(pltpu.async_copy(
                        table_hbm.at[idx_v.at[j, pl.ds(off, n)]],
                        rows_v.at[j, pl.ds(off, n)],
                        sem))
            for cp in copies:
                cp.wait()
            pltpu.sync_copy(rows_v, out_hbm.at[pl.ds(r, RCHUNK)])
            return carry

        lax.fori_loop(0, n_chunks, body, 0)

    return gather_kernel


def kernel(event, table):
    n, s = event.shape
    return _make_gather(n, s)(event, table)


# speedup vs baseline: 7.3530x; 1.5021x over previous
import functools

import jax
import jax.numpy as jnp
from jax import lax
from jax.experimental import pallas as pl
from jax.experimental.pallas import tpu as pltpu
from jax.experimental.pallas import tpu_sc as plsc

D_MODEL = 32
NC = 2
NS = 16
NW = NC * NS
JC = 8
IT = 128
L = 16


@functools.lru_cache(maxsize=None)
def _make_gather(N: int, S: int):
    itiles_per_w = N // (NW * IT)
    n_jc = S // JC
    mesh = plsc.VectorSubcoreMesh(core_axis_name="c", subcore_axis_name="s")

    @functools.partial(
        pl.kernel,
        mesh=mesh,
        compiler_params=pltpu.CompilerParams(use_tc_tiling_on_sc=False,
                                             needs_layout_passes=False),
        out_type=jax.ShapeDtypeStruct((S * 4, N // IT, 8 * IT), jnp.float32),
        scratch_types=[
            pltpu.VMEM((JC, IT), jnp.int32),
            pltpu.VMEM((JC * IT, D_MODEL), jnp.float32),
            pltpu.VMEM((JC * 4, 1, 8 * IT), jnp.float32),
            pltpu.SemaphoreType.DMA,
        ],
    )
    def gather_kernel(event_hbm, table_hbm, out_hbm, idx_v, rows_v, stage_v,
                      sem):
        wid = lax.axis_index("s") * NC + lax.axis_index("c")
        iota = lax.iota(jnp.int32, L)
        zero16 = iota * 0

        def jc_body(jc, it_carry):
            it, itile_abs = it_carry
            i0 = itile_abs * IT
            j0 = jc * JC
            pltpu.sync_copy(event_hbm.at[pl.ds(j0, JC), pl.ds(i0, IT)],
                            idx_v)
            copies = [
                pltpu.async_copy(
                    table_hbm.at[idx_v.at[m]],
                    rows_v.at[pl.ds(m * IT, IT)],
                    sem)
                for m in range(JC)
            ]
            for cp in copies:
                cp.wait()

            def j_body(j_loc, carry):
                row_base = iota + j_loc * IT
                d0_base = j_loc * 4
                for k0 in range(D_MODEL):
                    col_v = (iota + k0) & 31
                    d0_v = d0_base + (col_v >> 3)
                    d2_base = ((col_v & 7) << 7) + iota
                    for ib in range(0, IT, L):
                        x = plsc.load_gather(rows_v, [row_base + ib, col_v])
                        plsc.store_scatter(stage_v,
                                           [d0_v, zero16, d2_base + ib], x)
                return carry

            lax.fori_loop(0, JC, j_body, 0)
            pltpu.sync_copy(stage_v,
                            out_hbm.at[pl.ds(j0 * 4, JC * 4),
                                       pl.ds(itile_abs, 1)])
            return it_carry

        def it_body(it, carry):
            itile_abs = wid * itiles_per_w + it
            lax.fori_loop(0, n_jc, jc_body, (it, itile_abs))
            return carry

        lax.fori_loop(0, itiles_per_w, it_body, 0)

    return gather_kernel


def kernel(event, table):
    n, s = event.shape
    out3 = _make_gather(n, s)(event.T, table)
    out5 = out3.reshape(s, 4, n // IT, 8, IT)
    return out5.transpose(2, 4, 0, 1, 3).reshape(n, s, D_MODEL)


# speedup vs baseline: 8.2309x; 1.1194x over previous
import functools

import jax
import jax.numpy as jnp
from jax import lax
from jax.experimental import pallas as pl
from jax.experimental.pallas import tpu as pltpu
from jax.experimental.pallas import tpu_sc as plsc

D_MODEL = 32
NC = 2
NS = 16
NW = NC * NS
JC = 4
IT = 128
L = 16


@functools.lru_cache(maxsize=None)
def _make_gather(N: int, S: int):
    itiles_per_w = N // (NW * IT)
    n_jc = S // JC
    mesh = plsc.VectorSubcoreMesh(core_axis_name="c", subcore_axis_name="s")

    @functools.partial(
        pl.kernel,
        mesh=mesh,
        compiler_params=pltpu.CompilerParams(use_tc_tiling_on_sc=False,
                                             needs_layout_passes=False),
        out_type=jax.ShapeDtypeStruct((S * 4, N // IT, 8 * IT), jnp.float32),
        scratch_types=[
            pltpu.VMEM((JC, IT), jnp.int32),
            pltpu.VMEM((JC, IT), jnp.int32),
            pltpu.VMEM((JC * IT, D_MODEL), jnp.float32),
            pltpu.VMEM((JC * IT, D_MODEL), jnp.float32),
            pltpu.VMEM((JC * 4, 8 * IT), jnp.float32),
            pltpu.VMEM((JC * 4, 8 * IT), jnp.float32),
            pltpu.SemaphoreType.DMA,
            pltpu.SemaphoreType.DMA,
        ],
    )
    def gather_kernel(event_hbm, table_hbm, out_hbm, idx0, idx1, rows0,
                      rows1, st0, st1, semg, semo):
        wid = lax.axis_index("s") * NC + lax.axis_index("c")
        iota = lax.iota(jnp.int32, L)

        def load_and_fire(idx_v, rows_v, itile_abs, jc):
            pltpu.sync_copy(
                event_hbm.at[pl.ds(jc * JC, JC),
                             pl.ds(itile_abs * IT, IT)],
                idx_v)
            for m in range(JC):
                pltpu.async_copy(table_hbm.at[idx_v.at[m]],
                                 rows_v.at[pl.ds(m * IT, IT)], semg)

        def drain_gathers(idx_v, rows_v):
            for m in range(JC):
                pltpu.make_async_copy(table_hbm.at[idx_v.at[m]],
                                      rows_v.at[pl.ds(m * IT, IT)],
                                      semg).wait()

        def transpose(rows_v, st_v):
            def j_body(j_loc, c):
                row_base = iota + j_loc * IT
                d0_base = j_loc * 4

                def k_body(k0, c2):
                    col_v = (iota + k0) & 31
                    d0_v = d0_base + (col_v >> 3)
                    d2_base = ((col_v & 7) << 7) + iota
                    for ib in range(0, IT, L):
                        x = plsc.load_gather(rows_v,
                                             [row_base + ib, col_v])
                        plsc.store_scatter(st_v, [d0_v, d2_base + ib], x)
                    return c2

                lax.fori_loop(0, D_MODEL, k_body, 0)
                return c

            lax.fori_loop(0, JC, j_body, 0)

        def out_copy(st_v, itile_abs, jc, issue):
            dst = out_hbm.at[pl.ds(jc * JC * 4, JC * 4), itile_abs]
            if issue:
                pltpu.async_copy(st_v, dst, semo)
            else:
                pltpu.make_async_copy(st_v, dst, semo).wait()

        def it_body(it, carry):
            itile_abs = wid * itiles_per_w + it
            load_and_fire(idx0, rows0, itile_abs, 0)

            def pair_body(p, carry2):
                e = 2 * p
                o = 2 * p + 1
                drain_gathers(idx0, rows0)
                load_and_fire(idx1, rows1, itile_abs, o)

                @pl.when(p > 0)
                def _():
                    out_copy(st0, itile_abs, e - 2, issue=False)

                transpose(rows0, st0)
                out_copy(st0, itile_abs, e, issue=True)

                drain_gathers(idx1, rows1)

                @pl.when(o + 1 < n_jc)
                def _():
                    load_and_fire(idx0, rows0, itile_abs, o + 1)

                @pl.when(p > 0)
                def _():
                    out_copy(st1, itile_abs, o - 2, issue=False)

                transpose(rows1, st1)
                out_copy(st1, itile_abs, o, issue=True)
                return carry2

            lax.fori_loop(0, n_jc // 2, pair_body, 0)
            out_copy(st0, itile_abs, n_jc - 2, issue=False)
            out_copy(st1, itile_abs, n_jc - 1, issue=False)
            return carry

        lax.fori_loop(0, itiles_per_w, it_body, 0)

    return gather_kernel


def kernel(event, table):
    n, s = event.shape
    out3 = _make_gather(n, s)(event.T, table)
    out5 = out3.reshape(s, 4, n // IT, 8, IT)
    return out5.transpose(2, 4, 0, 1, 3).reshape(n, s, D_MODEL)


# speedup vs baseline: 8.3087x; 1.0095x over previous
import functools

import jax
import jax.numpy as jnp
from jax import lax
from jax.experimental import pallas as pl
from jax.experimental.pallas import tpu as pltpu
from jax.experimental.pallas import tpu_sc as plsc

D_MODEL = 32
NC = 2
NS = 16
NW = NC * NS
JC = 4
IT = 128
L = 16


@functools.lru_cache(maxsize=None)
def _make_gather(N: int, S: int):
    itiles_per_w = N // (NW * IT)
    n_jc = S // JC
    mesh = plsc.VectorSubcoreMesh(core_axis_name="c", subcore_axis_name="s")

    @functools.partial(
        pl.kernel,
        mesh=mesh,
        compiler_params=pltpu.CompilerParams(use_tc_tiling_on_sc=False,
                                             needs_layout_passes=False),
        out_type=jax.ShapeDtypeStruct((S * 4, N // IT, 8 * IT), jnp.float32),
        scratch_types=[
            pltpu.VMEM((JC, IT), jnp.int32),
            pltpu.VMEM((JC, IT), jnp.int32),
            pltpu.VMEM((JC * IT, D_MODEL), jnp.float32),
            pltpu.VMEM((JC * IT, D_MODEL), jnp.float32),
            pltpu.VMEM((JC * 4 * 8 * IT,), jnp.float32),
            pltpu.VMEM((JC * 4 * 8 * IT,), jnp.float32),
            pltpu.SemaphoreType.DMA,
            pltpu.SemaphoreType.DMA,
        ],
    )
    def gather_kernel(event_hbm, table_hbm, out_hbm, idx0, idx1, rows0,
                      rows1, st0, st1, semg, semo):
        wid = lax.axis_index("s") * NC + lax.axis_index("c")
        iota = lax.iota(jnp.int32, L)

        def load_and_fire(idx_v, rows_v, itile_abs, jc):
            pltpu.sync_copy(
                event_hbm.at[pl.ds(jc * JC, JC),
                             pl.ds(itile_abs * IT, IT)],
                idx_v)
            for m in range(JC):
                pltpu.async_copy(table_hbm.at[idx_v.at[m]],
                                 rows_v.at[pl.ds(m * IT, IT)], semg)

        def drain_gathers(idx_v, rows_v):
            for m in range(JC):
                pltpu.make_async_copy(table_hbm.at[idx_v.at[m]],
                                      rows_v.at[pl.ds(m * IT, IT)],
                                      semg).wait()

        def transpose(rows_v, st_v):
            def j_body(j_loc, c):
                row_base = iota + j_loc * IT
                dst0 = ((j_loc * 4 + (iota >> 3)) << 10) \
                    + ((iota & 7) << 7) + iota

                def k_body(k0, carry):
                    col_v, dst_v = carry
                    for ib in range(0, IT, L):
                        x = plsc.load_gather(rows_v,
                                             [row_base + ib, col_v])
                        plsc.store_scatter(st_v, [dst_v + ib], x)
                    wrap = col_v == 31
                    col_n = (col_v + 1) & 31
                    dst_n = dst_v + jnp.where(wrap, 128 - 4096, 128)
                    return (col_n, dst_n)

                lax.fori_loop(0, D_MODEL, k_body, (iota & 31, dst0))
                return c

            lax.fori_loop(0, JC, j_body, 0)

        def out_copy(st_v, itile_abs, jc, issue):
            for r in range(JC * 4):
                src = st_v.at[pl.ds(r * 8 * IT, 8 * IT)]
                dst = out_hbm.at[jc * JC * 4 + r, itile_abs]
                if issue:
                    pltpu.async_copy(src, dst, semo)
                else:
                    pltpu.make_async_copy(src, dst, semo).wait()

        def it_body(it, carry):
            itile_abs = wid * itiles_per_w + it
            load_and_fire(idx0, rows0, itile_abs, 0)

            def pair_body(p, carry2):
                e = 2 * p
                o = 2 * p + 1
                drain_gathers(idx0, rows0)
                load_and_fire(idx1, rows1, itile_abs, o)

                @pl.when(p > 0)
                def _():
                    out_copy(st0, itile_abs, e - 2, issue=False)

                transpose(rows0, st0)
                out_copy(st0, itile_abs, e, issue=True)

                drain_gathers(idx1, rows1)

                @pl.when(o + 1 < n_jc)
                def _():
                    load_and_fire(idx0, rows0, itile_abs, o + 1)

                @pl.when(p > 0)
                def _():
                    out_copy(st1, itile_abs, o - 2, issue=False)

                transpose(rows1, st1)
                out_copy(st1, itile_abs, o, issue=True)
                return carry2

            lax.fori_loop(0, n_jc // 2, pair_body, 0)
            out_copy(st0, itile_abs, n_jc - 2, issue=False)
            out_copy(st1, itile_abs, n_jc - 1, issue=False)
            return carry

        lax.fori_loop(0, itiles_per_w, it_body, 0)

    return gather_kernel


def kernel(event, table):
    n, s = event.shape
    out3 = _make_gather(n, s)(event.T, table)
    out5 = out3.reshape(s, 4, n // IT, 8, IT)
    return out5.transpose(2, 4, 0, 1, 3).reshape(n, s, D_MODEL)
